# baseline (speedup 1.0000x reference)
import jax
import jax.numpy as jnp
from jax import lax
from jax.experimental import pallas as pl
from jax.experimental.pallas import tpu as pltpu
from jax.experimental.pallas import tpu_sc as plsc

N = 10000
E = 160000
C = 32
NC = 1000
EC = 4000

NP = 10240
EPAD = 163840
KB = 128
CHB = EPAD // (32 * KB)
ROWS_B = NP // 16

NCP = 1024
ECP = 4096
KS = 128
KF = 32
ROWS_S = NCP // 16

_MESH = plsc.VectorSubcoreMesh(core_axis_name="c", subcore_axis_name="s")
_SC_PARAMS = pltpu.CompilerParams(needs_layout_passes=False,
                                  use_tc_tiling_on_sc=False)
_TC_PARAMS = pltpu.CompilerParams(vmem_limit_bytes=100 * 1024 * 1024)
_F32 = jnp.float32


def _leaky(x):
    return jnp.where(x >= 0, x, 0.01 * x)


def _dot(a, b):
    return jax.lax.dot_general(a, b, (((a.ndim - 1,), (0,)), ((), ())),
                               preferred_element_type=_F32)


def _zero_ref(ref):
    dims = ref.shape
    z = jnp.zeros((16,), _F32)
    per_row = dims[-1] // 16
    nrows = 1
    for d in dims[:-1]:
        nrows *= d

    def body(i, _):
        col = (i % per_row) * 16
        row = i // per_row
        idxs = []
        for d in reversed(dims[:-1]):
            idxs.append(row % d)
            row = row // d
        idxs.reverse()
        ref[(*idxs, pl.ds(col, 16))] = z
        return 0

    lax.fori_loop(0, nrows * per_row, body, 0)


def _seg_big(nparts, with_counts):
    NBUF = 4
    outs = [jax.ShapeDtypeStruct((nparts, 2, NP, 64), _F32)]
    scratch = [
        pltpu.VMEM((CHB, KB), jnp.int32),
        pltpu.VMEM((CHB, KB), jnp.int32),
        [pltpu.VMEM((KB, 64), _F32) for _ in range(NBUF)],
        pltpu.VMEM((KB, 64), _F32),
        pltpu.VMEM_SHARED((NP, 64), _F32),
        pltpu.SemaphoreType.DMA,
        pltpu.SemaphoreType.DMA,
    ]
    if with_counts:
        outs.append(jax.ShapeDtypeStruct((32, NP), _F32))
        scratch += [pltpu.VMEM((NP,), _F32)]

    def body(*refs):
        us = refs[:nparts]
        src_hbm, dst_hbm, out_hbm = refs[nparts:nparts + 3]
        rest = refs[nparts + 3:]
        if with_counts:
            cnt_out, sidx, didx, rows, zbuf, acc, semg, sems, cnt_v = rest
        else:
            sidx, didx, rows, zbuf, acc, semg, sems = rest
        c = lax.axis_index("c")
        s = lax.axis_index("s")
        wid = c * 16 + s

        pltpu.sync_copy(src_hbm.at[pl.ds(wid * CHB, CHB)], sidx)
        pltpu.sync_copy(dst_hbm.at[pl.ds(wid * CHB, CHB)], didx)
        _zero_ref(zbuf)
        if with_counts:
            _zero_ref(cnt_v)
            ones = jnp.ones((16,), _F32)
            for i in range(CHB):
                for j in range(KB // 16):
                    idx16 = didx[i, pl.ds(j * 16, 16)]
                    plsc.addupdate_scatter(cnt_v, [idx16], ones)
            pltpu.sync_copy(cnt_v, cnt_out.at[wid])

        for part in range(nparts):
            u_hbm = us[part]
            for k in range(ROWS_B // KB):
                pltpu.sync_copy(zbuf, acc.at[pl.ds(s * ROWS_B + k * KB, KB)])
            plsc.subcore_barrier()
            gh = [None] * CHB
            sh = [None] * CHB
            for i in range(min(NBUF, CHB)):
                gh[i] = pltpu.async_copy(u_hbm.at[sidx.at[i]],
                                         rows[i % NBUF], semg)
            for i in range(CHB):
                gh[i].wait()
                sh[i] = pltpu.async_copy(rows[i % NBUF], acc.at[didx.at[i]],
                                         sems, add=True)
                if i + NBUF < CHB:
                    sh[i].wait()
                    gh[i + NBUF] = pltpu.async_copy(
                        u_hbm.at[sidx.at[i + NBUF]], rows[i % NBUF], semg)
            for i in range(max(CHB - NBUF, 0), CHB):
                sh[i].wait()
            plsc.subcore_barrier()
            pltpu.sync_copy(acc.at[pl.ds(s * ROWS_B, ROWS_B)],
                            out_hbm.at[part, c, pl.ds(s * ROWS_B, ROWS_B)])
            plsc.subcore_barrier()

    return pl.kernel(body, out_type=outs, mesh=_MESH, scratch_types=scratch,
                     compiler_params=_SC_PARAMS)


def _seg_small():
    outs = jax.ShapeDtypeStruct((2, NCP, 64), _F32)
    scratch = [
        pltpu.VMEM((KS,), jnp.int32),
        pltpu.VMEM((KS,), jnp.int32),
        pltpu.VMEM((KS, 64), _F32),
        pltpu.VMEM((ROWS_S, 64), _F32),
        pltpu.VMEM_SHARED((NCP, 64), _F32),
        pltpu.SemaphoreType.DMA,
    ]

    def body(u_hbm, src_hbm, dst_hbm, out_hbm, sidx, didx, rows, zbuf, acc, sem):
        c = lax.axis_index("c")
        s = lax.axis_index("s")
        wid = c * 16 + s
        _zero_ref(zbuf)
        pltpu.sync_copy(zbuf, acc.at[pl.ds(s * ROWS_S, ROWS_S)])
        plsc.subcore_barrier()
        pltpu.sync_copy(src_hbm.at[wid], sidx)
        pltpu.async_copy(u_hbm.at[sidx], rows, sem).wait()
        pltpu.sync_copy(dst_hbm.at[wid], didx)
        pltpu.sync_copy(rows, acc.at[didx], add=True)
        plsc.subcore_barrier()
        pltpu.sync_copy(acc.at[pl.ds(s * ROWS_S, ROWS_S)],
                        out_hbm.at[c, pl.ds(s * ROWS_S, ROWS_S)])

    return pl.kernel(body, out_type=outs, mesh=_MESH, scratch_types=scratch,
                     compiler_params=_SC_PARAMS)


def _seg_fat(nparts):
    NBUF = 4
    NCH = ECP // KF // 16
    outs = jax.ShapeDtypeStruct((nparts, 4, NCP, 8, 64), _F32)
    scratch = [
        pltpu.VMEM((NCH, KF), jnp.int32),
        pltpu.VMEM((NCH, KF), jnp.int32),
        [pltpu.VMEM((KF, 8, 64), _F32) for _ in range(NBUF)],
        pltpu.VMEM((KF, 8, 64), _F32),
        pltpu.VMEM_SHARED((NCP, 8, 64), _F32),
        pltpu.SemaphoreType.DMA,
        pltpu.SemaphoreType.DMA,
    ]

    def body(*refs):
        us = refs[:nparts]
        src_hbm, dst_hbm, out_hbm = refs[nparts:nparts + 3]
        sidx, didx, rows, zbuf, acc, semg, sems = refs[nparts + 3:]
        c = lax.axis_index("c")
        s = lax.axis_index("s")
        pltpu.sync_copy(src_hbm.at[pl.ds(s * NCH, NCH)], sidx)
        pltpu.sync_copy(dst_hbm.at[pl.ds(s * NCH, NCH)], didx)
        _zero_ref(zbuf)
        for part in range(nparts):
            for p in range(2):
                slab = c * 2 + p
                for k in range(ROWS_S // KF):
                    pltpu.sync_copy(zbuf,
                                    acc.at[pl.ds(s * ROWS_S + k * KF, KF)])
                plsc.subcore_barrier()
                u_hbm = us[part].at[slab]
                gh = [None] * NCH
                sh = [None] * NCH
                for i in range(min(NBUF, NCH)):
                    gh[i] = pltpu.async_copy(u_hbm.at[sidx.at[i]],
                                             rows[i % NBUF], semg)
                for i in range(NCH):
                    gh[i].wait()
                    sh[i] = pltpu.async_copy(rows[i % NBUF],
                                             acc.at[didx.at[i]], sems,
                                             add=True)
                    if i + NBUF < NCH:
                        sh[i].wait()
                        gh[i + NBUF] = pltpu.async_copy(
                            u_hbm.at[sidx.at[i + NBUF]], rows[i % NBUF], semg)
                for i in range(max(NCH - NBUF, 0), NCH):
                    sh[i].wait()
                plsc.subcore_barrier()
                pltpu.sync_copy(acc.at[pl.ds(s * ROWS_S, ROWS_S)],
                                out_hbm.at[part, slab,
                                           pl.ds(s * ROWS_S, ROWS_S)])
                plsc.subcore_barrier()

    return pl.kernel(body, out_type=outs, mesh=_MESH, scratch_types=scratch,
                     compiler_params=_SC_PARAMS)


def _emb_sc():
    outs = jax.ShapeDtypeStruct((NP, 32), _F32)
    scratch = [
        pltpu.VMEM((KF,), jnp.int32),
        pltpu.VMEM((KF, 32), _F32),
        pltpu.SemaphoreType.DMA,
    ]

    def body(emb_hbm, opc_hbm, out_hbm, idv, rows, sem):
        c = lax.axis_index("c")
        s = lax.axis_index("s")
        wid = c * 16 + s
        nch = NP // KF // 32

        def chunk(i, _):
            row = wid * nch + i
            pltpu.sync_copy(opc_hbm.at[row], idv)
            pltpu.async_copy(emb_hbm.at[idv], rows, sem).wait()
            pltpu.sync_copy(rows, out_hbm.at[pl.ds(row * KF, KF)])
            return 0

        lax.fori_loop(0, nch, chunk, 0)

    return pl.kernel(body, out_type=outs, mesh=_MESH, scratch_types=scratch,
                     compiler_params=_SC_PARAMS)


def _misc_sc():
    outs = [
        jax.ShapeDtypeStruct((NCP, 64), _F32),
        jax.ShapeDtypeStruct((NCP, 64), _F32),
        jax.ShapeDtypeStruct((32, NCP), _F32),
    ]
    scratch = [
        pltpu.VMEM((KF,), jnp.int32),
        pltpu.VMEM((KF, 64), _F32),
        pltpu.VMEM((KS,), jnp.int32),
        pltpu.VMEM((NCP,), _F32),
        pltpu.SemaphoreType.DMA,
    ]

    def body(agg_hbm, x3_hbm, ids_hbm, cdst_hbm, nb_out, xc_out, cnt_out,
             idv, rows, didx, cnt_v, sem):
        c = lax.axis_index("c")
        s = lax.axis_index("s")
        wid = c * 16 + s
        _zero_ref(cnt_v)

        pltpu.sync_copy(ids_hbm.at[wid], idv)
        pltpu.async_copy(agg_hbm.at[idv], rows, sem).wait()
        pltpu.sync_copy(rows, nb_out.at[pl.ds(wid * KF, KF)])
        pltpu.async_copy(x3_hbm.at[idv], rows, sem).wait()
        pltpu.sync_copy(rows, xc_out.at[pl.ds(wid * KF, KF)])

        pltpu.sync_copy(cdst_hbm.at[wid], didx)
        ones = jnp.ones((16,), _F32)
        for j in range(KS // 16):
            idx16 = didx[pl.ds(j * 16, 16)]
            plsc.addupdate_scatter(cnt_v, [idx16], ones)
        pltpu.sync_copy(cnt_v, cnt_out.at[wid])

    return pl.kernel(body, out_type=outs, mesh=_MESH, scratch_types=scratch,
                     compiler_params=_SC_PARAMS)


_RB = 1000


def _row_spec(*dims):
    return pl.BlockSpec((_RB,) + dims, lambda i: (i,) + (0,) * len(dims))


def _full_spec(sh):
    return pl.BlockSpec(sh, lambda i: tuple(0 for _ in sh))


_PSPEC = pl.BlockSpec((2, _RB, 64), lambda i: (0, i, 0))


def _tc_b1(Pa, Pb, Pc, cnt, x0a, x0b, x0c, Wl, Wr, bl):
    def body(pa, pb, pc, cn, xa, xb, xc_, wl, wr, bl_, x_o, ic_o):
        ic = jnp.maximum(jnp.sum(cn[:], axis=1, keepdims=True), 1.0)
        ic_o[:] = ic
        acc = _dot((pa[0] + pa[1]) / ic, wl[0:64, :])
        acc += _dot((pb[0] + pb[1]) / ic, wl[64:128, :])
        acc += _dot((pc[0] + pc[1]) / ic, wl[128:192, :])
        acc += _dot(xa[:], wr[0:64, :])
        acc += _dot(xb[:], wr[64:128, :])
        acc += _dot(xc_[:], wr[128:192, :])
        x_o[:] = _leaky(acc + bl_[0, :])

    wspec = _full_spec((192, 64))
    return pl.pallas_call(
        body, out_shape=[jax.ShapeDtypeStruct((N, 64), _F32),
                         jax.ShapeDtypeStruct((N, 1), _F32)],
        grid=(N // _RB,),
        in_specs=[_PSPEC, _PSPEC, _PSPEC, _row_spec(32), _row_spec(64),
                  _row_spec(64), _row_spec(64), wspec, wspec,
                  _full_spec((1, 64))],
        out_specs=[_row_spec(64), _row_spec(1)],
        compiler_params=_TC_PARAMS)(Pa, Pb, Pc, cnt, x0a, x0b, x0c, Wl, Wr, bl)


def _tc_b(P, invc, x, Wl, Wr, bl):
    def body(p, ic, xx, wl, wr, bl_, x_o):
        m = (p[0] + p[1]) / ic[:]
        x_o[:] = _leaky(_dot(m, wl[:]) + _dot(xx[:], wr[:]) + bl_[0, :])

    return pl.pallas_call(
        body, out_shape=jax.ShapeDtypeStruct((N, 64), _F32),
        grid=(N // _RB,),
        in_specs=[_PSPEC, _row_spec(1), _row_spec(64), _full_spec((64, 64)),
                  _full_spec((64, 64)), _full_spec((1, 64))],
        out_specs=_row_spec(64),
        compiler_params=_TC_PARAMS)(P, invc, x, Wl, Wr, bl)


def _tc_c(P4, invc):
    def body(p, ic, agg_o):
        agg_o[:] = (p[0] + p[1]) / ic[:]
    return pl.pallas_call(
        body, out_shape=jax.ShapeDtypeStruct((N, 64), _F32),
        grid=(N // _RB,),
        in_specs=[_PSPEC, _row_spec(1)],
        out_specs=_row_spec(64),
        compiler_params=_TC_PARAMS)(P4, invc)


def _tc_d(Ps, ccnt, nb, Wl, Wr, bl):
    def body(p, cn, nbv, wl, wr, bl_, x_o):
        cc = jnp.maximum(jnp.sum(cn[:], axis=1, keepdims=True), 1.0)
        m = (p[0] + p[1]) / cc
        x_o[:] = _leaky(_dot(m, wl[:]) + _dot(nbv[:], wr[:]) + bl_[0, :])
    return pl.pallas_call(
        body, out_shape=jax.ShapeDtypeStruct((NCP, 64), _F32),
        compiler_params=_TC_PARAMS)(Ps, ccnt, nb, Wl, Wr, bl)


def _tc_e(nb3, xc, nct, prjW, prjb):
    def body(nb, x3, nc, pw, pb, m0_o, m1_o, m2_o):
        nbv = nb[:]
        xcv = x3[:]
        ncv = _leaky(_dot(nc[:], pw[:]) + pb[0, :])
        sq = (jnp.sum(nbv * nbv, axis=1, keepdims=True)
              + jnp.sum(xcv * xcv, axis=1, keepdims=True))
        nrm2 = sq[None, :, :, None] + jnp.sum(ncv * ncv, axis=3, keepdims=True)
        scale = 1.0 / jnp.maximum(jnp.sqrt(nrm2), 1e-12)
        m0_o[:] = scale * nbv[None, :, None, :]
        m1_o[:] = scale * xcv[None, :, None, :]
        m2_o[:] = scale * ncv

    slab18 = pl.BlockSpec((1, NCP, 8, 18), lambda i: (i, 0, 0, 0))
    slab64 = pl.BlockSpec((1, NCP, 8, 64), lambda i: (i, 0, 0, 0))
    sh = jax.ShapeDtypeStruct((4, NCP, 8, 64), _F32)
    return pl.pallas_call(
        body, out_shape=[sh, sh, sh],
        grid=(4,),
        in_specs=[_full_spec((NCP, 64)), _full_spec((NCP, 64)), slab18,
                  _full_spec((18, 64)), _full_spec((1, 64))],
        out_specs=[slab64, slab64, slab64],
        compiler_params=_TC_PARAMS)(nb3, xc, nct, prjW, prjb)


_SLAB64 = pl.BlockSpec((1, NCP, 8, 64), lambda i: (i, 0, 0, 0))
_CCSPEC = pl.BlockSpec((NCP, 32), lambda i: (0, 0))


def _tc_f1(Pm0, Pm1, Pm2, ccnt, m0, m1, m2, Wl, Wr, bl):
    def body(p0, p1, p2, cn, a0, a1, a2, wl, wr, bl_, h_o):
        cc = jnp.maximum(jnp.sum(cn[:], axis=1, keepdims=True), 1.0)
        cc4 = cc[None, :, :, None]
        acc = _dot(p0[:] / cc4, wl[0:64, :])
        acc += _dot(p1[:] / cc4, wl[64:128, :])
        acc += _dot(p2[:] / cc4, wl[128:192, :])
        acc += _dot(a0[:], wr[0:64, :])
        acc += _dot(a1[:], wr[64:128, :])
        acc += _dot(a2[:], wr[128:192, :])
        h_o[:] = _leaky(acc + bl_[0, :])

    qs = pl.BlockSpec((1, NCP // 4, 8, 64), lambda i, j: (i, j, 0, 0))
    ccq = pl.BlockSpec((NCP // 4, 32), lambda i, j: (j, 0))
    wq = pl.BlockSpec((192, 64), lambda i, j: (0, 0))
    blq = pl.BlockSpec((1, 64), lambda i, j: (0, 0))
    return pl.pallas_call(
        body, out_shape=jax.ShapeDtypeStruct((4, NCP, 8, 64), _F32),
        grid=(4, 4),
        in_specs=[qs, qs, qs, ccq, qs, qs, qs, wq, wq, blq],
        out_specs=qs,
        compiler_params=_TC_PARAMS)(Pm0, Pm1, Pm2, ccnt, m0, m1, m2,
                                    Wl, Wr, bl)


def _tc_f(Ph, ccnt, h, Wl, Wr, bl):
    def body(p, cn, hv, wl, wr, bl_, h_o):
        cc = jnp.maximum(jnp.sum(cn[:], axis=1, keepdims=True), 1.0)
        m = p[:] / cc[None, :, :, None]
        h_o[:] = _leaky(_dot(m, wl[:]) + _dot(hv[:], wr[:]) + bl_[0, :])

    return pl.pallas_call(
        body, out_shape=jax.ShapeDtypeStruct((4, NCP, 8, 64), _F32),
        grid=(4,),
        in_specs=[_SLAB64, _CCSPEC, _SLAB64, _full_spec((64, 64)),
                  _full_spec((64, 64)), _full_spec((1, 64))],
        out_specs=_SLAB64,
        compiler_params=_TC_PARAMS)(Ph, ccnt, h, Wl, Wr, bl)


def _tc_g(Ph, ccnt, h, Wl, Wr, bl, d1, d2, d3):
    def body(p, cn, hv, wl, wr, bl_, w1, w2, w3, y_o):
        cc = jnp.maximum(jnp.sum(cn[:], axis=1, keepdims=True), 1.0)
        m = p[:] / cc[None, :, :, None]
        h3 = _leaky(_dot(m, wl[:]) + _dot(hv[:], wr[:]) + bl_[0, :])
        pooled = jnp.sum(h3[:, :NC], axis=1) / NC
        y = _leaky(_dot(pooled, w1[:]))
        y = _leaky(_dot(y, w2[:]))
        y_o[:] = _dot(y, w3[:])

    return pl.pallas_call(
        body, out_shape=jax.ShapeDtypeStruct((4, 8, 1), _F32),
        compiler_params=_TC_PARAMS)(Ph, ccnt, h, Wl, Wr, bl, d1, d2, d3)


def kernel(node_feat, node_opcode, edge_index, node_config_feat,
           node_config_ids, config_edge_index, params):
    p = params
    gnn = p["node_gnn"]
    cgnn = p["cfg_nbr_gnn"]
    fgnn = p["cfg_gnn"]

    src = edge_index[0].astype(jnp.int32)
    dst = edge_index[1].astype(jnp.int32)
    pad = EPAD - E
    src_p = jnp.concatenate([src, jnp.zeros((pad,), jnp.int32)]).reshape(-1, KB)
    dst_p = jnp.concatenate([dst, jnp.full((pad,), NP - 8, jnp.int32)]
                            ).reshape(-1, KB)

    csrc = config_edge_index[0].astype(jnp.int32)
    cdst = config_edge_index[1].astype(jnp.int32)
    cpad = ECP - EC
    csrc_s = jnp.concatenate([csrc, jnp.zeros((cpad,), jnp.int32)])
    cdst_s = jnp.concatenate([cdst, jnp.full((cpad,), NCP - 8, jnp.int32)])
    csrc_ks = csrc_s.reshape(-1, KS)
    cdst_ks = cdst_s.reshape(-1, KS)
    csrc_kf = csrc_s.reshape(-1, KF)
    cdst_kf = cdst_s.reshape(-1, KF)

    ids_p = jnp.concatenate([node_config_ids.astype(jnp.int32),
                             jnp.zeros((NCP - NC,), jnp.int32)]).reshape(-1, KF)
    nct = jnp.pad(node_config_feat.transpose(1, 0, 2),
                  ((0, NCP - NC), (0, 0), (0, 0))
                  ).reshape(NCP, 4, 8, 18).transpose(1, 0, 2, 3)
    opc_p = jnp.concatenate([node_opcode.astype(jnp.int32),
                             jnp.zeros((NP - N,), jnp.int32)]).reshape(-1, KF)
    b2 = lambda b: b.reshape(1, -1)
    pad192 = lambda w: jnp.pad(w, ((0, 192 - w.shape[0]), (0, 0)))

    er = _emb_sc()(p["embedding"], opc_p)
    x0a = node_feat[:, 0:64]
    x0b = node_feat[:, 64:128]
    x0c = jnp.concatenate([node_feat[:, 128:140], er[:N],
                           jnp.zeros((N, 20), _F32)], axis=1)
    Pabc, cnt = _seg_big(3, True)(x0a, x0b, x0c, src_p, dst_p)
    x, invc = _tc_b1(Pabc[0], Pabc[1], Pabc[2], cnt.T, x0a, x0b, x0c,
                     pad192(gnn[0]["Wl"]), pad192(gnn[0]["Wr"]),
                     b2(gnn[0]["bl"]))
    P = _seg_big(1, False)(x, src_p, dst_p)[0][0]
    x = _tc_b(P, invc, x, gnn[1]["Wl"], gnn[1]["Wr"], b2(gnn[1]["bl"]))
    P = _seg_big(1, False)(x, src_p, dst_p)[0][0]
    x3 = _tc_b(P, invc, x, gnn[2]["Wl"], gnn[2]["Wr"], b2(gnn[2]["bl"]))
    P4 = _seg_big(1, False)(x3, src_p, dst_p)[0][0]
    agg = _tc_c(P4, invc)

    nb, xc, ccnt = _misc_sc()(agg, x3, ids_p, cdst_ks)
    cct = ccnt.T

    for lay in cgnn:
        Ps = _seg_small()(nb, csrc_ks, cdst_ks)
        nb = _tc_d(Ps, cct, nb, lay["Wl"], lay["Wr"], b2(lay["bl"]))

    m0, m1, m2 = _tc_e(nb, xc, nct, p["prj_W"], b2(p["prj_b"]))
    Pm = _seg_fat(3)(m0, m1, m2, csrc_kf, cdst_kf)
    h = _tc_f1(Pm[0], Pm[1], Pm[2], cct, m0, m1, m2,
               pad192(fgnn[0]["Wl"]), pad192(fgnn[0]["Wr"]), b2(fgnn[0]["bl"]))
    Ph = _seg_fat(1)(h, csrc_kf, cdst_kf)[0]
    h = _tc_f(Ph, cct, h, fgnn[1]["Wl"], fgnn[1]["Wr"], b2(fgnn[1]["bl"]))
    Ph = _seg_fat(1)(h, csrc_kf, cdst_kf)[0]
    y = _tc_g(Ph, cct, h, fgnn[2]["Wl"], fgnn[2]["Wr"], b2(fgnn[2]["bl"]),
              p["d1"], p["d2"], p["d3"])
    return y.reshape(-1)


# speedup vs baseline: 1.1187x; 1.1187x over previous
import jax
import jax.numpy as jnp
from jax import lax
from jax.experimental import pallas as pl
from jax.experimental.pallas import tpu as pltpu
from jax.experimental.pallas import tpu_sc as plsc

N = 10000
E = 160000
C = 32
NC = 1000
EC = 4000

NP = 10240
EPAD = 163840
KB = 128
CHB = EPAD // (32 * KB)
ROWS_B = NP // 16

NCP = 1024
ECP = 4096
KS = 128
KF = 32
ROWS_S = NCP // 16

_MESH = plsc.VectorSubcoreMesh(core_axis_name="c", subcore_axis_name="s")
_SC_PARAMS = pltpu.CompilerParams(needs_layout_passes=False,
                                  use_tc_tiling_on_sc=False)
_TC_PARAMS = pltpu.CompilerParams(vmem_limit_bytes=100 * 1024 * 1024)
_F32 = jnp.float32


def _leaky(x):
    return jnp.where(x >= 0, x, 0.01 * x)


def _dot(a, b):
    return jax.lax.dot_general(a, b, (((a.ndim - 1,), (0,)), ((), ())),
                               preferred_element_type=_F32)


def _zero_ref(ref):
    dims = ref.shape
    z = jnp.zeros((16,), _F32)
    per_row = dims[-1] // 16
    nrows = 1
    for d in dims[:-1]:
        nrows *= d

    def body(i, _):
        col = (i % per_row) * 16
        row = i // per_row
        idxs = []
        for d in reversed(dims[:-1]):
            idxs.append(row % d)
            row = row // d
        idxs.reverse()
        ref[(*idxs, pl.ds(col, 16))] = z
        return 0

    lax.fori_loop(0, nrows * per_row, body, 0)


def _seg_big(nparts, with_counts):
    NBUF = 4
    outs = [jax.ShapeDtypeStruct((nparts, 2, NP, 64), _F32)]
    scratch = [
        pltpu.VMEM((CHB, KB), jnp.int32),
        pltpu.VMEM((CHB, KB), jnp.int32),
        [pltpu.VMEM((KB, 64), _F32) for _ in range(NBUF)],
        pltpu.VMEM((KB, 64), _F32),
        pltpu.VMEM_SHARED((NP, 64), _F32),
        pltpu.SemaphoreType.DMA,
        pltpu.SemaphoreType.DMA,
    ]
    if with_counts:
        outs.append(jax.ShapeDtypeStruct((32, NP), _F32))
        scratch += [pltpu.VMEM((NP,), _F32)]

    def body(*refs):
        us = refs[:nparts]
        src_hbm, dst_hbm, out_hbm = refs[nparts:nparts + 3]
        rest = refs[nparts + 3:]
        if with_counts:
            cnt_out, sidx, didx, rows, zbuf, acc, semg, sems, cnt_v = rest
        else:
            sidx, didx, rows, zbuf, acc, semg, sems = rest
        c = lax.axis_index("c")
        s = lax.axis_index("s")
        wid = c * 16 + s

        pltpu.sync_copy(src_hbm.at[pl.ds(wid * CHB, CHB)], sidx)
        pltpu.sync_copy(dst_hbm.at[pl.ds(wid * CHB, CHB)], didx)
        _zero_ref(zbuf)
        if with_counts:
            _zero_ref(cnt_v)
            ones = jnp.ones((16,), _F32)
            for i in range(CHB):
                for j in range(KB // 16):
                    idx16 = didx[i, pl.ds(j * 16, 16)]
                    plsc.addupdate_scatter(cnt_v, [idx16], ones)
            pltpu.sync_copy(cnt_v, cnt_out.at[wid])

        for part in range(nparts):
            u_hbm = us[part]
            for k in range(ROWS_B // KB):
                pltpu.sync_copy(zbuf, acc.at[pl.ds(s * ROWS_B + k * KB, KB)])
            plsc.subcore_barrier()
            gh = [None] * CHB
            sh = [None] * CHB
            for i in range(min(NBUF, CHB)):
                gh[i] = pltpu.async_copy(u_hbm.at[sidx.at[i]],
                                         rows[i % NBUF], semg)
            for i in range(CHB):
                gh[i].wait()
                sh[i] = pltpu.async_copy(rows[i % NBUF], acc.at[didx.at[i]],
                                         sems, add=True)
                if i + NBUF < CHB:
                    sh[i].wait()
                    gh[i + NBUF] = pltpu.async_copy(
                        u_hbm.at[sidx.at[i + NBUF]], rows[i % NBUF], semg)
            for i in range(max(CHB - NBUF, 0), CHB):
                sh[i].wait()
            plsc.subcore_barrier()
            pltpu.sync_copy(acc.at[pl.ds(s * ROWS_B, ROWS_B)],
                            out_hbm.at[part, c, pl.ds(s * ROWS_B, ROWS_B)])
            plsc.subcore_barrier()

    return pl.kernel(body, out_type=outs, mesh=_MESH, scratch_types=scratch,
                     compiler_params=_SC_PARAMS)


def _seg_small():
    outs = jax.ShapeDtypeStruct((2, NCP, 64), _F32)
    scratch = [
        pltpu.VMEM((KS,), jnp.int32),
        pltpu.VMEM((KS,), jnp.int32),
        pltpu.VMEM((KS, 64), _F32),
        pltpu.VMEM((ROWS_S, 64), _F32),
        pltpu.VMEM_SHARED((NCP, 64), _F32),
        pltpu.SemaphoreType.DMA,
    ]

    def body(u_hbm, src_hbm, dst_hbm, out_hbm, sidx, didx, rows, zbuf, acc, sem):
        c = lax.axis_index("c")
        s = lax.axis_index("s")
        wid = c * 16 + s
        _zero_ref(zbuf)
        pltpu.sync_copy(zbuf, acc.at[pl.ds(s * ROWS_S, ROWS_S)])
        plsc.subcore_barrier()
        pltpu.sync_copy(src_hbm.at[wid], sidx)
        pltpu.async_copy(u_hbm.at[sidx], rows, sem).wait()
        pltpu.sync_copy(dst_hbm.at[wid], didx)
        pltpu.sync_copy(rows, acc.at[didx], add=True)
        plsc.subcore_barrier()
        pltpu.sync_copy(acc.at[pl.ds(s * ROWS_S, ROWS_S)],
                        out_hbm.at[c, pl.ds(s * ROWS_S, ROWS_S)])

    return pl.kernel(body, out_type=outs, mesh=_MESH, scratch_types=scratch,
                     compiler_params=_SC_PARAMS)


def _seg_fat(nparts):
    NBUF = 4
    NCH = ECP // KF // 16
    outs = jax.ShapeDtypeStruct((nparts, 4, NCP, 8, 64), _F32)
    scratch = [
        pltpu.VMEM((NCH, KF), jnp.int32),
        pltpu.VMEM((NCH, KF), jnp.int32),
        [pltpu.VMEM((KF, 8, 64), _F32) for _ in range(NBUF)],
        pltpu.VMEM((KF, 8, 64), _F32),
        pltpu.VMEM_SHARED((NCP, 8, 64), _F32),
        pltpu.SemaphoreType.DMA,
        pltpu.SemaphoreType.DMA,
    ]

    def body(*refs):
        us = refs[:nparts]
        src_hbm, dst_hbm, out_hbm = refs[nparts:nparts + 3]
        sidx, didx, rows, zbuf, acc, semg, sems = refs[nparts + 3:]
        c = lax.axis_index("c")
        s = lax.axis_index("s")
        pltpu.sync_copy(src_hbm.at[pl.ds(s * NCH, NCH)], sidx)
        pltpu.sync_copy(dst_hbm.at[pl.ds(s * NCH, NCH)], didx)
        _zero_ref(zbuf)
        for part in range(nparts):
            for p in range(2):
                slab = c * 2 + p
                for k in range(ROWS_S // KF):
                    pltpu.sync_copy(zbuf,
                                    acc.at[pl.ds(s * ROWS_S + k * KF, KF)])
                plsc.subcore_barrier()
                u_hbm = us[part].at[slab]
                gh = [None] * NCH
                sh = [None] * NCH
                for i in range(min(NBUF, NCH)):
                    gh[i] = pltpu.async_copy(u_hbm.at[sidx.at[i]],
                                             rows[i % NBUF], semg)
                for i in range(NCH):
                    gh[i].wait()
                    sh[i] = pltpu.async_copy(rows[i % NBUF],
                                             acc.at[didx.at[i]], sems,
                                             add=True)
                    if i + NBUF < NCH:
                        sh[i].wait()
                        gh[i + NBUF] = pltpu.async_copy(
                            u_hbm.at[sidx.at[i + NBUF]], rows[i % NBUF], semg)
                for i in range(max(NCH - NBUF, 0), NCH):
                    sh[i].wait()
                plsc.subcore_barrier()
                pltpu.sync_copy(acc.at[pl.ds(s * ROWS_S, ROWS_S)],
                                out_hbm.at[part, slab,
                                           pl.ds(s * ROWS_S, ROWS_S)])
                plsc.subcore_barrier()

    return pl.kernel(body, out_type=outs, mesh=_MESH, scratch_types=scratch,
                     compiler_params=_SC_PARAMS)


def _emb_sc():
    outs = jax.ShapeDtypeStruct((NP, 32), _F32)
    scratch = [
        pltpu.VMEM((KF,), jnp.int32),
        pltpu.VMEM((KF, 32), _F32),
        pltpu.SemaphoreType.DMA,
    ]

    def body(emb_hbm, opc_hbm, out_hbm, idv, rows, sem):
        c = lax.axis_index("c")
        s = lax.axis_index("s")
        wid = c * 16 + s
        nch = NP // KF // 32

        def chunk(i, _):
            row = wid * nch + i
            pltpu.sync_copy(opc_hbm.at[row], idv)
            pltpu.async_copy(emb_hbm.at[idv], rows, sem).wait()
            pltpu.sync_copy(rows, out_hbm.at[pl.ds(row * KF, KF)])
            return 0

        lax.fori_loop(0, nch, chunk, 0)

    return pl.kernel(body, out_type=outs, mesh=_MESH, scratch_types=scratch,
                     compiler_params=_SC_PARAMS)


def _misc_sc():
    outs = [
        jax.ShapeDtypeStruct((NCP, 64), _F32),
        jax.ShapeDtypeStruct((NCP, 64), _F32),
        jax.ShapeDtypeStruct((32, NCP), _F32),
    ]
    scratch = [
        pltpu.VMEM((KF,), jnp.int32),
        pltpu.VMEM((KF, 64), _F32),
        pltpu.VMEM((KS,), jnp.int32),
        pltpu.VMEM((NCP,), _F32),
        pltpu.SemaphoreType.DMA,
    ]

    def body(agg_hbm, x3_hbm, ids_hbm, cdst_hbm, nb_out, xc_out, cnt_out,
             idv, rows, didx, cnt_v, sem):
        c = lax.axis_index("c")
        s = lax.axis_index("s")
        wid = c * 16 + s
        _zero_ref(cnt_v)

        pltpu.sync_copy(ids_hbm.at[wid], idv)
        pltpu.async_copy(agg_hbm.at[idv], rows, sem).wait()
        pltpu.sync_copy(rows, nb_out.at[pl.ds(wid * KF, KF)])
        pltpu.async_copy(x3_hbm.at[idv], rows, sem).wait()
        pltpu.sync_copy(rows, xc_out.at[pl.ds(wid * KF, KF)])

        pltpu.sync_copy(cdst_hbm.at[wid], didx)
        ones = jnp.ones((16,), _F32)
        for j in range(KS // 16):
            idx16 = didx[pl.ds(j * 16, 16)]
            plsc.addupdate_scatter(cnt_v, [idx16], ones)
        pltpu.sync_copy(cnt_v, cnt_out.at[wid])

    return pl.kernel(body, out_type=outs, mesh=_MESH, scratch_types=scratch,
                     compiler_params=_SC_PARAMS)


_RB = 1000


def _row_spec(*dims):
    return pl.BlockSpec((_RB,) + dims, lambda i: (i,) + (0,) * len(dims))


def _full_spec(sh):
    return pl.BlockSpec(sh, lambda i: tuple(0 for _ in sh))


_PSPEC = pl.BlockSpec((2, _RB, 64), lambda i: (0, i, 0))


def _tc_b1(Pa, Pb, Pc, cnt, x0a, x0b, x0c, Wl, Wr, bl):
    def body(pa, pb, pc, cn, xa, xb, xc_, wl, wr, bl_, x_o, ic_o):
        ic = jnp.maximum(jnp.sum(cn[:], axis=1, keepdims=True), 1.0)
        ic_o[:] = ic
        acc = _dot((pa[0] + pa[1]) / ic, wl[0:64, :])
        acc += _dot((pb[0] + pb[1]) / ic, wl[64:128, :])
        acc += _dot((pc[0] + pc[1]) / ic, wl[128:192, :])
        acc += _dot(xa[:], wr[0:64, :])
        acc += _dot(xb[:], wr[64:128, :])
        acc += _dot(xc_[:], wr[128:192, :])
        x_o[:] = _leaky(acc + bl_[0, :])

    wspec = _full_spec((192, 64))
    return pl.pallas_call(
        body, out_shape=[jax.ShapeDtypeStruct((N, 64), _F32),
                         jax.ShapeDtypeStruct((N, 1), _F32)],
        grid=(N // _RB,),
        in_specs=[_PSPEC, _PSPEC, _PSPEC, _row_spec(32), _row_spec(64),
                  _row_spec(64), _row_spec(64), wspec, wspec,
                  _full_spec((1, 64))],
        out_specs=[_row_spec(64), _row_spec(1)],
        compiler_params=_TC_PARAMS)(Pa, Pb, Pc, cnt, x0a, x0b, x0c, Wl, Wr, bl)


def _tc_b(P, invc, x, Wl, Wr, bl):
    def body(p, ic, xx, wl, wr, bl_, x_o):
        m = (p[0] + p[1]) / ic[:]
        x_o[:] = _leaky(_dot(m, wl[:]) + _dot(xx[:], wr[:]) + bl_[0, :])

    return pl.pallas_call(
        body, out_shape=jax.ShapeDtypeStruct((N, 64), _F32),
        grid=(N // _RB,),
        in_specs=[_PSPEC, _row_spec(1), _row_spec(64), _full_spec((64, 64)),
                  _full_spec((64, 64)), _full_spec((1, 64))],
        out_specs=_row_spec(64),
        compiler_params=_TC_PARAMS)(P, invc, x, Wl, Wr, bl)


def _tc_c(P4, invc):
    def body(p, ic, agg_o):
        agg_o[:] = (p[0] + p[1]) / ic[:]
    return pl.pallas_call(
        body, out_shape=jax.ShapeDtypeStruct((N, 64), _F32),
        grid=(N // _RB,),
        in_specs=[_PSPEC, _row_spec(1)],
        out_specs=_row_spec(64),
        compiler_params=_TC_PARAMS)(P4, invc)


def _tc_d(Ps, ccnt, nb, Wl, Wr, bl):
    def body(p, cn, nbv, wl, wr, bl_, x_o):
        cc = jnp.maximum(jnp.sum(cn[:], axis=1, keepdims=True), 1.0)
        m = (p[0] + p[1]) / cc
        x_o[:] = _leaky(_dot(m, wl[:]) + _dot(nbv[:], wr[:]) + bl_[0, :])
    return pl.pallas_call(
        body, out_shape=jax.ShapeDtypeStruct((NCP, 64), _F32),
        compiler_params=_TC_PARAMS)(Ps, ccnt, nb, Wl, Wr, bl)


def _tc_e(nb3, xc, nct, prjW, prjb):
    def body(nb, x3, nc, pw, pb, m0_o, m1_o, m2_o):
        nbv = nb[:]
        xcv = x3[:]
        ncv = _leaky(_dot(nc[:], pw[:]) + pb[0, :])
        sq = (jnp.sum(nbv * nbv, axis=1, keepdims=True)
              + jnp.sum(xcv * xcv, axis=1, keepdims=True))
        nrm2 = sq[None, :, :, None] + jnp.sum(ncv * ncv, axis=3, keepdims=True)
        scale = 1.0 / jnp.maximum(jnp.sqrt(nrm2), 1e-12)
        m0_o[:] = scale * nbv[None, :, None, :]
        m1_o[:] = scale * xcv[None, :, None, :]
        m2_o[:] = scale * ncv

    slab18 = pl.BlockSpec((1, NCP, 8, 18), lambda i: (i, 0, 0, 0))
    slab64 = pl.BlockSpec((1, NCP, 8, 64), lambda i: (i, 0, 0, 0))
    sh = jax.ShapeDtypeStruct((4, NCP, 8, 64), _F32)
    return pl.pallas_call(
        body, out_shape=[sh, sh, sh],
        grid=(4,),
        in_specs=[_full_spec((NCP, 64)), _full_spec((NCP, 64)), slab18,
                  _full_spec((18, 64)), _full_spec((1, 64))],
        out_specs=[slab64, slab64, slab64],
        compiler_params=_TC_PARAMS)(nb3, xc, nct, prjW, prjb)


_SLAB64 = pl.BlockSpec((1, NCP, 8, 64), lambda i: (i, 0, 0, 0))
_CCSPEC = pl.BlockSpec((NCP, 32), lambda i: (0, 0))


def _tc_f1(Pm0, Pm1, Pm2, ccnt, m0, m1, m2, Wl, Wr, bl):
    def body(p0, p1, p2, cn, a0, a1, a2, wl, wr, bl_, h_o):
        cc = jnp.maximum(jnp.sum(cn[:], axis=1, keepdims=True), 1.0)
        cc4 = cc[None, :, :, None]
        acc = _dot(p0[:] / cc4, wl[0:64, :])
        acc += _dot(p1[:] / cc4, wl[64:128, :])
        acc += _dot(p2[:] / cc4, wl[128:192, :])
        acc += _dot(a0[:], wr[0:64, :])
        acc += _dot(a1[:], wr[64:128, :])
        acc += _dot(a2[:], wr[128:192, :])
        h_o[:] = _leaky(acc + bl_[0, :])

    qs = pl.BlockSpec((1, NCP // 4, 8, 64), lambda i, j: (i, j, 0, 0))
    ccq = pl.BlockSpec((NCP // 4, 32), lambda i, j: (j, 0))
    wq = pl.BlockSpec((192, 64), lambda i, j: (0, 0))
    blq = pl.BlockSpec((1, 64), lambda i, j: (0, 0))
    return pl.pallas_call(
        body, out_shape=jax.ShapeDtypeStruct((4, NCP, 8, 64), _F32),
        grid=(4, 4),
        in_specs=[qs, qs, qs, ccq, qs, qs, qs, wq, wq, blq],
        out_specs=qs,
        compiler_params=_TC_PARAMS)(Pm0, Pm1, Pm2, ccnt, m0, m1, m2,
                                    Wl, Wr, bl)


def _tc_f(Ph, ccnt, h, Wl, Wr, bl):
    def body(p, cn, hv, wl, wr, bl_, h_o):
        cc = jnp.maximum(jnp.sum(cn[:], axis=1, keepdims=True), 1.0)
        m = p[:] / cc[None, :, :, None]
        h_o[:] = _leaky(_dot(m, wl[:]) + _dot(hv[:], wr[:]) + bl_[0, :])

    return pl.pallas_call(
        body, out_shape=jax.ShapeDtypeStruct((4, NCP, 8, 64), _F32),
        grid=(4,),
        in_specs=[_SLAB64, _CCSPEC, _SLAB64, _full_spec((64, 64)),
                  _full_spec((64, 64)), _full_spec((1, 64))],
        out_specs=_SLAB64,
        compiler_params=_TC_PARAMS)(Ph, ccnt, h, Wl, Wr, bl)


def _tc_g(Ph, ccnt, h, Wl, Wr, bl, d1, d2, d3):
    def body(p, cn, hv, wl, wr, bl_, w1, w2, w3, y_o):
        cc = jnp.maximum(jnp.sum(cn[:], axis=1, keepdims=True), 1.0)
        m = p[:] / cc[None, :, :, None]
        h3 = _leaky(_dot(m, wl[:]) + _dot(hv[:], wr[:]) + bl_[0, :])
        pooled = jnp.sum(h3[:, :NC], axis=1) / NC
        y = _leaky(_dot(pooled, w1[:]))
        y = _leaky(_dot(y, w2[:]))
        y_o[:] = _dot(y, w3[:])

    return pl.pallas_call(
        body, out_shape=jax.ShapeDtypeStruct((4, 8, 1), _F32),
        compiler_params=_TC_PARAMS)(Ph, ccnt, h, Wl, Wr, bl, d1, d2, d3)


def kernel(node_feat, node_opcode, edge_index, node_config_feat,
           node_config_ids, config_edge_index, params):
    p = params
    gnn = p["node_gnn"]
    cgnn = p["cfg_nbr_gnn"]
    fgnn = p["cfg_gnn"]

    src = edge_index[0].astype(jnp.int32)
    dst = edge_index[1].astype(jnp.int32)
    pad = EPAD - E
    src_p = jnp.concatenate([src, jnp.zeros((pad,), jnp.int32)]).reshape(-1, KB)
    dst_p = jnp.concatenate([dst, jnp.full((pad,), NP - 8, jnp.int32)]
                            ).reshape(-1, KB)

    csrc = config_edge_index[0].astype(jnp.int32)
    cdst = config_edge_index[1].astype(jnp.int32)
    cpad = ECP - EC
    csrc_s = jnp.concatenate([csrc, jnp.zeros((cpad,), jnp.int32)])
    cdst_s = jnp.concatenate([cdst, jnp.full((cpad,), NCP - 8, jnp.int32)])
    csrc_ks = csrc_s.reshape(-1, KS)
    cdst_ks = cdst_s.reshape(-1, KS)
    csrc_kf = csrc_s.reshape(-1, KF)
    cdst_kf = cdst_s.reshape(-1, KF)

    ids_p = jnp.concatenate([node_config_ids.astype(jnp.int32),
                             jnp.zeros((NCP - NC,), jnp.int32)]).reshape(-1, KF)
    nct = jnp.pad(node_config_feat.transpose(1, 0, 2),
                  ((0, NCP - NC), (0, 0), (0, 0))
                  ).reshape(NCP, 4, 8, 18).transpose(1, 0, 2, 3)
    opc_p = jnp.concatenate([node_opcode.astype(jnp.int32),
                             jnp.zeros((NP - N,), jnp.int32)]).reshape(-1, KF)
    b2 = lambda b: b.reshape(1, -1)
    pad192 = lambda w: jnp.pad(w, ((0, 192 - w.shape[0]), (0, 0)))

    er = _emb_sc()(p["embedding"], opc_p)
    x0a = node_feat[:, 0:64]
    x0b = node_feat[:, 64:128]
    x0c = jnp.concatenate([node_feat[:, 128:140], er[:N],
                           jnp.zeros((N, 20), _F32)], axis=1)
    Pa, cnt = _seg_big(1, True)(x0a, src_p, dst_p)
    Pb = _seg_big(1, False)(x0b, src_p, dst_p)[0]
    Pc = _seg_big(1, False)(x0c, src_p, dst_p)[0]
    x, invc = _tc_b1(Pa[0], Pb[0], Pc[0], cnt.T, x0a, x0b, x0c,
                     pad192(gnn[0]["Wl"]), pad192(gnn[0]["Wr"]),
                     b2(gnn[0]["bl"]))
    P = _seg_big(1, False)(x, src_p, dst_p)[0][0]
    x = _tc_b(P, invc, x, gnn[1]["Wl"], gnn[1]["Wr"], b2(gnn[1]["bl"]))
    P = _seg_big(1, False)(x, src_p, dst_p)[0][0]
    x3 = _tc_b(P, invc, x, gnn[2]["Wl"], gnn[2]["Wr"], b2(gnn[2]["bl"]))
    P4 = _seg_big(1, False)(x3, src_p, dst_p)[0][0]
    agg = _tc_c(P4, invc)

    nb, xc, ccnt = _misc_sc()(agg, x3, ids_p, cdst_ks)
    cct = ccnt.T

    for lay in cgnn:
        Ps = _seg_small()(nb, csrc_ks, cdst_ks)
        nb = _tc_d(Ps, cct, nb, lay["Wl"], lay["Wr"], b2(lay["bl"]))

    m0, m1, m2 = _tc_e(nb, xc, nct, p["prj_W"], b2(p["prj_b"]))
    Pm0 = _seg_fat(1)(m0, csrc_kf, cdst_kf)[0]
    Pm1 = _seg_fat(1)(m1, csrc_kf, cdst_kf)[0]
    Pm2 = _seg_fat(1)(m2, csrc_kf, cdst_kf)[0]
    h = _tc_f1(Pm0, Pm1, Pm2, cct, m0, m1, m2,
               pad192(fgnn[0]["Wl"]), pad192(fgnn[0]["Wr"]), b2(fgnn[0]["bl"]))
    Ph = _seg_fat(1)(h, csrc_kf, cdst_kf)[0]
    h = _tc_f(Ph, cct, h, fgnn[1]["Wl"], fgnn[1]["Wr"], b2(fgnn[1]["bl"]))
    Ph = _seg_fat(1)(h, csrc_kf, cdst_kf)[0]
    y = _tc_g(Ph, cct, h, fgnn[2]["Wl"], fgnn[2]["Wr"], b2(fgnn[2]["bl"]),
              p["d1"], p["d2"], p["d3"])
    return y.reshape(-1)
